# Initial kernel scaffold; baseline (speedup 1.0000x reference)
import jax
import jax.numpy as jnp
from jax.experimental import pallas as pl


def kernel(src_v, src_f, tgt_v, tgt_f):
    raise NotImplementedError("write your pallas kernel here")


# speedup vs baseline: 5.4910x; 5.4910x over previous
import functools

import jax
import jax.numpy as jnp
from jax import lax
from jax.experimental import pallas as pl
from jax.experimental.pallas import tpu as pltpu

_NUM_QUERY = 20000
_STD = 0.05
_QB = 256
_FB = 512


def _sample_surface(faces, vs, count, key):
    v0 = vs[faces[:, 0]]
    v1 = vs[faces[:, 1]]
    v2 = vs[faces[:, 2]]
    fn = jnp.cross(v1 - v0, v2 - v0)
    areas = jnp.linalg.norm(fn, axis=1)
    weights = 0.5 * areas
    probs = weights / jnp.sum(weights)
    k1, k2 = jax.random.split(key)
    face_index = jax.random.choice(k1, faces.shape[0], shape=(count,), p=probs)
    tri_o = v0[face_index]
    tv1 = (v1 - v0)[face_index]
    tv2 = (v2 - v0)[face_index]
    rl = jax.random.uniform(k2, (count, 2, 1), dtype=vs.dtype)
    test = jnp.sum(rl, axis=1).reshape(-1) > 1.0
    rl = jnp.where(test[:, None, None], rl - 1.0, rl)
    rl = jnp.abs(rl)
    samples = tv1 * rl[:, 0] + tv2 * rl[:, 1] + tri_o
    return samples


def _face_consts(v, f, f_pad):
    a = v[f[:, 0]]
    b = v[f[:, 1]]
    c = v[f[:, 2]]
    ab = b - a
    ac = c - a
    abab = jnp.sum(ab * ab, -1)
    abac = jnp.sum(ab * ac, -1)
    acac = jnp.sum(ac * ac, -1)
    bcbc = abab - 2.0 * abac + acac
    g = lambda x: jnp.where(jnp.abs(x) < 1e-12, 1.0, x)
    n = f.shape[0]
    rows = jnp.stack([
        a[:, 0], a[:, 1], a[:, 2],
        ab[:, 0], ab[:, 1], ab[:, 2],
        ac[:, 0], ac[:, 1], ac[:, 2],
        abab, abac, acac,
        1.0 / g(abab), 1.0 / g(acac), 1.0 / g(bcbc),
        jnp.zeros((n,), jnp.float32),
    ])
    pad = f_pad - n
    if pad:
        s = jnp.zeros((16, pad), jnp.float32)
        s = s.at[0:3].set(1e6)
        s = s.at[12:15].set(1.0)
        rows = jnp.concatenate([rows, s], axis=1)
    return rows


def _cp_body(q_ref, fc_ref, o_ref, bd, bcx, bcy, bcz, *, nf):
    j = pl.program_id(2)
    f = fc_ref[0]
    q = q_ref[...]
    px = q[:, 0:1]
    py = q[:, 1:2]
    pz = q[:, 2:3]
    ax, ay, az = f[0:1], f[1:2], f[2:3]
    abx, aby, abz = f[3:4], f[4:5], f[5:6]
    acx, acy, acz = f[6:7], f[7:8], f[8:9]
    abab, abac, acac = f[9:10], f[10:11], f[11:12]
    r_ab, r_ac, r_bc = f[12:13], f[13:14], f[14:15]

    apx = px - ax
    apy = py - ay
    apz = pz - az
    d1 = abx * apx + aby * apy + abz * apz
    d2 = acx * apx + acy * apy + acz * apz
    d3 = d1 - abab
    d4 = d2 - abac
    d5 = d1 - abac
    d6 = d2 - acac
    vc = d1 * d4 - d3 * d2
    vb = d5 * d2 - d1 * d6
    va = d3 * d6 - d5 * d4
    den = va + vb + vc
    rden = 1.0 / jnp.where(jnp.abs(den) < 1e-12, 1.0, den)
    w2 = vb * rden
    w3 = vc * rden
    e1 = d4 - d3
    e2 = d5 - d6
    w_bc = e1 * r_bc
    cond = (va <= 0.0) & (e1 >= 0.0) & (e2 >= 0.0)
    w2 = jnp.where(cond, 1.0 - w_bc, w2)
    w3 = jnp.where(cond, w_bc, w3)
    w_ac = d2 * r_ac
    cond = (vb <= 0.0) & (d2 >= 0.0) & (d6 <= 0.0)
    w2 = jnp.where(cond, 0.0, w2)
    w3 = jnp.where(cond, w_ac, w3)
    cond = (d6 >= 0.0) & (d5 <= d6)
    w2 = jnp.where(cond, 0.0, w2)
    w3 = jnp.where(cond, 1.0, w3)
    v_ab = d1 * r_ab
    cond = (vc <= 0.0) & (d1 >= 0.0) & (d3 <= 0.0)
    w2 = jnp.where(cond, v_ab, w2)
    w3 = jnp.where(cond, 0.0, w3)
    cond = (d3 >= 0.0) & (d4 <= d3)
    w2 = jnp.where(cond, 1.0, w2)
    w3 = jnp.where(cond, 0.0, w3)
    cond = (d1 <= 0.0) & (d2 <= 0.0)
    w2 = jnp.where(cond, 0.0, w2)
    w3 = jnp.where(cond, 0.0, w3)

    cx = ax + w2 * abx + w3 * acx
    cy = ay + w2 * aby + w3 * acy
    cz = az + w2 * abz + w3 * acz
    dx = px - cx
    dy = py - cy
    dz = pz - cz
    dd = dx * dx + dy * dy + dz * dz

    @pl.when(j == 0)
    def _():
        bd[...] = dd
        bcx[...] = cx
        bcy[...] = cy
        bcz[...] = cz

    @pl.when(j > 0)
    def _():
        m = dd < bd[...]
        bd[...] = jnp.where(m, dd, bd[...])
        bcx[...] = jnp.where(m, cx, bcx[...])
        bcy[...] = jnp.where(m, cy, bcy[...])
        bcz[...] = jnp.where(m, cz, bcz[...])

    @pl.when(j == nf - 1)
    def _():
        bdv = bd[...]
        mn = jnp.min(bdv, axis=1)
        li = jnp.argmin(bdv, axis=1)
        oh = (lax.broadcasted_iota(jnp.int32, bdv.shape, 1)
              == li[:, None]).astype(jnp.float32)
        ocx = jnp.sum(oh * bcx[...], axis=1)
        ocy = jnp.sum(oh * bcy[...], axis=1)
        ocz = jnp.sum(oh * bcz[...], axis=1)
        o_ref[0] = jnp.stack([ocx, ocy, ocz, mn], axis=1)


@jax.jit
def _closest(queries, fcs):
    q_pad, _ = queries.shape
    f_pad = fcs.shape[-1]
    nq = q_pad // _QB
    nf = f_pad // _FB
    return pl.pallas_call(
        functools.partial(_cp_body, nf=nf),
        grid=(2, nq, nf),
        in_specs=[
            pl.BlockSpec((_QB, 3), lambda m, i, j: (i, 0)),
            pl.BlockSpec((1, 16, _FB), lambda m, i, j: (m, 0, j)),
        ],
        out_specs=pl.BlockSpec((1, _QB, 4), lambda m, i, j: (m, i, 0)),
        out_shape=jax.ShapeDtypeStruct((2, q_pad, 4), jnp.float32),
        scratch_shapes=[pltpu.VMEM((_QB, _FB), jnp.float32)] * 4,
        compiler_params=pltpu.CompilerParams(
            dimension_semantics=("parallel", "parallel", "arbitrary")),
    )(queries, fcs)


def kernel(src_v, src_f, tgt_v, tgt_f):
    key = jax.random.key(42)
    k_s, k_n = jax.random.split(key)
    qp = _sample_surface(tgt_f, tgt_v, _NUM_QUERY, k_s)
    qp = qp + jax.random.normal(k_n, qp.shape, dtype=qp.dtype) * _STD
    sf1 = src_v[src_f[:, 0]]
    sf2 = src_v[src_f[:, 1]]
    sf3 = src_v[src_f[:, 2]]
    src_center = (sf1 + sf2 + sf3) / 3.0
    query = lax.stop_gradient(jnp.concatenate([qp, src_center], axis=0))
    q = query.shape[0]
    q_pad = -(-q // _QB) * _QB
    f_pad = -(-src_f.shape[0] // _FB) * _FB
    qpad = jnp.concatenate(
        [query, jnp.zeros((q_pad - q, 3), query.dtype)], axis=0)
    fcs = jnp.stack([_face_consts(src_v, src_f, f_pad),
                     _face_consts(tgt_v, tgt_f, f_pad)])
    out = _closest(qpad, fcs)
    closest_src = out[0, :q, 0:3]
    closest_tgt = out[1, :q, 0:3]
    dir_src = query - closest_src
    udf_src = jnp.linalg.norm(dir_src + 1e-10, axis=-1, keepdims=True)
    geo_src = jnp.concatenate([dir_src, udf_src], axis=1)
    dir_tgt = query - closest_tgt
    udf_tgt = jnp.linalg.norm(dir_tgt + 1e-10, axis=-1, keepdims=True)
    geo_tgt = jnp.concatenate([dir_tgt, udf_tgt], axis=1)
    return jnp.mean(jnp.abs(geo_src - geo_tgt)) * 4.0
